# speedup vs baseline: 3.0099x; 3.0099x over previous
import functools

import jax
import jax.numpy as jnp
from jax import lax
from jax.experimental import pallas as pl
from jax.experimental.pallas import tpu as pltpu

N = 100000
D = 128
H = 128
NB = 64
BLK = 2000
GRID = N // BLK


def _gate_body(h_ref, w1_ref, b1_ref, w2t_ref, b2_ref, w_ref, m_ref, msc):
    i = pl.program_id(0)
    act = jnp.dot(h_ref[...], w1_ref[...],
                  preferred_element_type=jnp.float32) + b1_ref[...]
    act = act * jax.nn.sigmoid(act)
    w = jnp.sum(act * w2t_ref[...], axis=1, keepdims=True) + b2_ref[0, 0]
    w_ref[...] = w
    bm = jnp.max(w)
    prev = jnp.where(i == 0, -jnp.inf, msc[0, 0])
    msc[0, 0] = jnp.maximum(prev, bm)

    @pl.when(i == GRID - 1)
    def _():
        m_ref[...] = jnp.full((1, 16), msc[0, 0], dtype=jnp.float32)


def _pool_body(h_ref, w_ref, b_ref, m_ref, out_ref, num_sc, den_sc):
    i = pl.program_id(0)

    @pl.when(i == 0)
    def _():
        num_sc[...] = jnp.zeros_like(num_sc)
        den_sc[...] = jnp.zeros_like(den_sc)

    e = jnp.exp(w_ref[...] - m_ref[0, 0])
    seg = lax.broadcasted_iota(jnp.int32, (1, NB), 1)
    oh = (b_ref[...] == seg).astype(jnp.float32)
    eh = e * h_ref[...]
    num_sc[...] += lax.dot_general(
        oh, eh, (((0,), (0,)), ((), ())),
        preferred_element_type=jnp.float32,
        precision=lax.Precision.HIGHEST)
    den_sc[...] += lax.dot_general(
        e, oh, (((0,), (0,)), ((), ())),
        preferred_element_type=jnp.float32,
        precision=lax.Precision.HIGHEST)

    @pl.when(i == GRID - 1)
    def _():
        r = lax.broadcasted_iota(jnp.int32, (NB, NB), 0)
        c = lax.broadcasted_iota(jnp.int32, (NB, NB), 1)
        dcol = jnp.sum(jnp.where(r == c, den_sc[...], 0.0),
                       axis=1, keepdims=True)
        out_ref[...] = num_sc[...] / (dcol + 1e-6)


@jax.jit
def kernel(h, batch, W1, b1, W2, b2):
    b1r = b1.reshape(1, H)
    w2t = W2.reshape(1, H)
    b2r = b2.reshape(1, 1)
    bi32 = batch.astype(jnp.int32).reshape(N, 1)

    w, m = pl.pallas_call(
        _gate_body,
        grid=(GRID,),
        in_specs=[
            pl.BlockSpec((BLK, D), lambda i: (i, 0)),
            pl.BlockSpec((D, H), lambda i: (0, 0)),
            pl.BlockSpec((1, H), lambda i: (0, 0)),
            pl.BlockSpec((1, H), lambda i: (0, 0)),
            pl.BlockSpec((1, 1), lambda i: (0, 0)),
        ],
        out_specs=[
            pl.BlockSpec((BLK, 1), lambda i: (i, 0)),
            pl.BlockSpec((1, 16), lambda i: (0, 0)),
        ],
        out_shape=[
            jax.ShapeDtypeStruct((N, 1), jnp.float32),
            jax.ShapeDtypeStruct((1, 16), jnp.float32),
        ],
        scratch_shapes=[pltpu.SMEM((1, 1), jnp.float32)],
    )(h, W1, b1r, w2t, b2r)

    out = pl.pallas_call(
        _pool_body,
        grid=(GRID,),
        in_specs=[
            pl.BlockSpec((BLK, D), lambda i: (i, 0)),
            pl.BlockSpec((BLK, 1), lambda i: (i, 0)),
            pl.BlockSpec((BLK, 1), lambda i: (i, 0)),
            pl.BlockSpec((1, 16), lambda i: (0, 0)),
        ],
        out_specs=pl.BlockSpec((NB, D), lambda i: (0, 0)),
        out_shape=jax.ShapeDtypeStruct((NB, D), jnp.float32),
        scratch_shapes=[
            pltpu.VMEM((NB, D), jnp.float32),
            pltpu.VMEM((1, NB), jnp.float32),
        ],
    )(h, w, bi32, m)
    return out
